# baseline (speedup 1.0000x reference)
import functools

import jax
import jax.numpy as jnp
from jax import lax
from jax.experimental import pallas as pl
from jax.experimental.pallas import tpu as pltpu
from jax.experimental.pallas import tpu_sc as plsc

B, N, C, H, W = 4, 6, 64, 40, 80
HW = H * W
PTS = 20000
PTSP = 20480
NWARP = B * N
NCORE, NSUB = 2, 16
NWORK = NCORE * NSUB
CG = 8
NCG = C // CG
PARTS = 2
PPTS = PTSP // PARTS
GROUPS = PPTS // 16
NTASK = B * NCG * PARTS
TPW = NTASK // NWORK


def _sc_warp(images_flat, px, py):
    mesh = plsc.VectorSubcoreMesh(core_axis_name="c", subcore_axis_name="s")

    @functools.partial(
        pl.kernel,
        out_type=jax.ShapeDtypeStruct((B, C, PTSP), jnp.float32),
        mesh=mesh,
        compiler_params=pltpu.CompilerParams(
            use_tc_tiling_on_sc=False, needs_layout_passes=False,
            disable_bounds_checks=True),
        scratch_types=[
            pltpu.VMEM((CG, HW), jnp.float32),
            pltpu.VMEM((CG, PPTS), jnp.float32),
            pltpu.VMEM((PPTS,), jnp.float32),
            pltpu.VMEM((PPTS,), jnp.float32),
            pltpu.SMEM((GROUPS,), jnp.int32),
        ],
    )
    def k(img_hbm, px_hbm, py_hbm, out_hbm, img_v, acc_v, px_v, py_v, flg_v):
        wid = lax.axis_index("s") * NCORE + lax.axis_index("c")
        neginf = jnp.full((16,), -jnp.inf, jnp.float32)

        def cam(n, b, cg, p0):
            wrp = b * N + n
            pltpu.sync_copy(
                img_hbm.at[pl.ds(wrp * C + cg * CG, CG), :], img_v)
            pltpu.sync_copy(px_hbm.at[pl.ds(wrp * PTSP + p0, PPTS)], px_v)
            pltpu.sync_copy(py_hbm.at[pl.ds(wrp * PTSP + p0, PPTS)], py_v)

            def group(g, _):
                off = g * 16
                pxv = jnp.clip(px_v[pl.ds(off, 16)], -4.0e4, 4.0e4)
                pyv = jnp.clip(py_v[pl.ds(off, 16)], -4.0e4, 4.0e4)
                xt = pxv.astype(jnp.int32)
                yt = pyv.astype(jnp.int32)
                xtf = xt.astype(jnp.float32)
                ytf = yt.astype(jnp.float32)
                x0f = jnp.where(pxv < xtf, xtf - 1.0, xtf)
                y0f = jnp.where(pyv < ytf, ytf - 1.0, ytf)
                x0 = jnp.where(pxv < xtf, xt - 1, xt)
                y0 = jnp.where(pyv < ytf, yt - 1, yt)
                inb = (x0 >= 0) & (x0 <= W - 2) & (y0 >= 0) & (y0 <= H - 2)

                def sample(_):
                    wx1 = pxv - x0f
                    wx0 = (x0f + 1.0) - pxv
                    m = jnp.where(inb, 1.0, 0.0).astype(jnp.float32)
                    wy1 = (pyv - y0f) * m
                    wy0 = ((y0f + 1.0) - pyv) * m
                    w00 = wx0 * wy0
                    w01 = wx0 * wy1
                    w10 = wx1 * wy0
                    w11 = wx1 * wy1
                    base = jnp.where(inb, y0 * W + x0, 0)
                    b01 = base + W
                    b10 = base + 1
                    b11 = base + W + 1
                    for c in range(CG):
                        v00 = plsc.load_gather(img_v.at[c], [base])
                        v01 = plsc.load_gather(img_v.at[c], [b01])
                        v10 = plsc.load_gather(img_v.at[c], [b10])
                        v11 = plsc.load_gather(img_v.at[c], [b11])
                        val = (w00 * v00 + w01 * v01) + (w10 * v10 + w11 * v11)
                        acc_v[c, pl.ds(off, 16)] = jnp.maximum(
                            acc_v[c, pl.ds(off, 16)], val)
                    return 0

                def skip(_):
                    flg_v[g] = 1
                    return 0

                lax.cond(jnp.any(inb), sample, skip, 0)
                return 0

            lax.fori_loop(0, GROUPS, group, 0)

        def task_body(t, _):
            tid = wid * TPW + t
            part = lax.rem(tid, PARTS)
            rest = lax.div(tid, PARTS)
            cg = lax.rem(rest, NCG)
            b = lax.div(rest, NCG)
            p0 = part * PPTS

            def init(g, _):
                off = g * 16
                for c in range(CG):
                    acc_v[c, pl.ds(off, 16)] = neginf
                flg_v[g] = 0
                return 0

            lax.fori_loop(0, GROUPS, init, 0)

            def cams(n, _):
                cam(n, b, cg, p0)
                return 0

            lax.fori_loop(0, N, cams, 0)

            def finish(g, _):
                off = g * 16
                zero_or_ninf = jnp.where(flg_v[g] > 0,
                                         jnp.float32(0.0),
                                         jnp.float32(-jnp.inf))
                for c in range(CG):
                    acc_v[c, pl.ds(off, 16)] = jnp.maximum(
                        acc_v[c, pl.ds(off, 16)], zero_or_ninf)
                return 0

            lax.fori_loop(0, GROUPS, finish, 0)
            pltpu.sync_copy(
                acc_v, out_hbm.at[b, pl.ds(cg * CG, CG), pl.ds(p0, PPTS)])
            return 0

        lax.fori_loop(0, TPW, task_body, 0)

    return k(images_flat, px, py)


def _rotation_from_euler(rolls, pitchs, yaws):
    si, sj, sk = jnp.sin(rolls), jnp.sin(pitchs), jnp.sin(yaws)
    ci, cj, ck = jnp.cos(rolls), jnp.cos(pitchs), jnp.cos(yaws)
    cc, cs = ci * ck, ci * sk
    sc, ss = si * ck, si * sk
    zeros = jnp.zeros_like(si)
    ones = jnp.ones_like(si)
    row0 = jnp.stack([cj * ck, sj * sc - cs, sj * cc + ss, zeros], axis=-1)
    row1 = jnp.stack([cj * sk, sj * ss + cc, sj * cs - sc, zeros], axis=-1)
    row2 = jnp.stack([-sj, cj * si, cj * ci, zeros], axis=-1)
    row3 = jnp.stack([zeros, zeros, zeros, ones], axis=-1)
    return jnp.stack([row0, row1, row2, row3], axis=1)


def _pix_coords(Ks, RTs, translation, yaw_roll_pitch):
    zs = translation[:, 2]
    rolls = yaw_roll_pitch[:, 1]
    pitchs = yaw_roll_pitch[:, 2]
    xl = jnp.linspace(-60.0, 60.0, 200)
    yl = jnp.linspace(-30.0, 30.0, 100)
    Yg, Xg = jnp.meshgrid(yl, xl, indexing='ij')
    x = jnp.broadcast_to(Xg.flatten()[None, :], (B, PTS))
    y = jnp.broadcast_to(Yg.flatten()[None, :], (B, PTS))
    z = jnp.ones_like(x) * zs[:, None]
    d = jnp.ones_like(x)
    coords = jnp.stack([x, y, z, d], axis=1)
    R = _rotation_from_euler(pitchs, rolls, jnp.zeros_like(rolls))
    planes = R @ coords
    planes = jnp.tile(planes, (N, 1, 1))
    P = (Ks @ RTs).reshape(-1, 4, 4)
    pix = P @ planes
    py = pix[:, 0] + H / 2.0
    px = pix[:, 2] - W / 8.0
    return px, py


def kernel(images, Ks, RTs, translation, yaw_roll_pitch):
    px, py = _pix_coords(Ks, RTs, translation, yaw_roll_pitch)
    px = jnp.pad(px, ((0, 0), (0, PTSP - PTS)),
                 constant_values=1.0e9).reshape(-1)
    py = jnp.pad(py, ((0, 0), (0, PTSP - PTS)),
                 constant_values=1.0e9).reshape(-1)
    images_flat = images.reshape(NWARP * C, HW)
    out = _sc_warp(images_flat, px, py)
    return out[:, :, :PTS].reshape(B, C, 100, 200)


# speedup vs baseline: 1.0485x; 1.0485x over previous
import functools

import jax
import jax.numpy as jnp
from jax import lax
from jax.experimental import pallas as pl
from jax.experimental.pallas import tpu as pltpu
from jax.experimental.pallas import tpu_sc as plsc

B, N, C, H, W = 4, 6, 64, 40, 80
HW = H * W
PTS = 20000
PTSP = 20480
NWARP = B * N
NCORE, NSUB = 2, 16
NWORK = NCORE * NSUB
CG = 8
NCG = C // CG
PARTS = 2
PPTS = PTSP // PARTS
GROUPS = PPTS // 16
NTASK = B * NCG * PARTS
TPW = NTASK // NWORK


def _sc_warp(images_flat, px, py):
    mesh = plsc.VectorSubcoreMesh(core_axis_name="c", subcore_axis_name="s")

    @functools.partial(
        pl.kernel,
        out_type=jax.ShapeDtypeStruct((B, C, PTSP), jnp.float32),
        mesh=mesh,
        compiler_params=pltpu.CompilerParams(
            use_tc_tiling_on_sc=False, needs_layout_passes=False,
            disable_bounds_checks=True),
        scratch_types=[
            pltpu.VMEM((CG * HW,), jnp.float32),
            pltpu.VMEM((CG, PPTS), jnp.float32),
            pltpu.VMEM((PPTS,), jnp.float32),
            pltpu.VMEM((PPTS,), jnp.float32),
            pltpu.SMEM((GROUPS,), jnp.int32),
        ],
    )
    def k(img_hbm, px_hbm, py_hbm, out_hbm, img_v, acc_v, px_v, py_v, flg_v):
        wid = lax.axis_index("s") * NCORE + lax.axis_index("c")
        neginf = jnp.full((16,), -jnp.inf, jnp.float32)

        def cam(n, b, cg, p0):
            wrp = b * N + n
            pltpu.sync_copy(
                img_hbm.at[pl.ds((wrp * C + cg * CG) * HW, CG * HW)], img_v)
            pltpu.sync_copy(px_hbm.at[pl.ds(wrp * PTSP + p0, PPTS)], px_v)
            pltpu.sync_copy(py_hbm.at[pl.ds(wrp * PTSP + p0, PPTS)], py_v)

            def group(g, _):
                off = g * 16
                pxv = jnp.clip(px_v[pl.ds(off, 16)], -4.0e4, 4.0e4)
                pyv = jnp.clip(py_v[pl.ds(off, 16)], -4.0e4, 4.0e4)
                xt = pxv.astype(jnp.int32)
                yt = pyv.astype(jnp.int32)
                x0 = jnp.where(pxv < xt.astype(jnp.float32), xt - 1, xt)
                y0 = jnp.where(pyv < yt.astype(jnp.float32), yt - 1, yt)
                inb = (x0 >= 0) & (x0 <= W - 2) & (y0 >= 0) & (y0 <= H - 2)

                def sample(_):
                    x0f = x0.astype(jnp.float32)
                    y0f = y0.astype(jnp.float32)
                    wx1 = pxv - x0f
                    wx0 = (x0f + 1.0) - pxv
                    m = jnp.where(inb, 1.0, 0.0).astype(jnp.float32)
                    wy1 = (pyv - y0f) * m
                    wy0 = ((y0f + 1.0) - pyv) * m
                    w00 = wx0 * wy0
                    w01 = wx0 * wy1
                    w10 = wx1 * wy0
                    w11 = wx1 * wy1
                    base = jnp.where(inb, y0 * W + x0, 0)
                    b01 = base + W
                    b10 = base + 1
                    b11 = base + W + 1
                    for c in range(CG):
                        coff = c * HW
                        v00 = plsc.load_gather(img_v, [base + coff])
                        v01 = plsc.load_gather(img_v, [b01 + coff])
                        v10 = plsc.load_gather(img_v, [b10 + coff])
                        v11 = plsc.load_gather(img_v, [b11 + coff])
                        val = (w00 * v00 + w01 * v01) + (w10 * v10 + w11 * v11)
                        acc_v[c, pl.ds(off, 16)] = jnp.maximum(
                            acc_v[c, pl.ds(off, 16)], val)
                    return 0

                def skip(_):
                    flg_v[g] = 1
                    return 0

                lax.cond(jnp.any(inb), sample, skip, 0)
                return 0

            lax.fori_loop(0, GROUPS, group, 0)

        def task_body(t, _):
            tid = wid * TPW + t
            part = lax.rem(tid, PARTS)
            rest = lax.div(tid, PARTS)
            cg = lax.rem(rest, NCG)
            b = lax.div(rest, NCG)
            p0 = part * PPTS

            def init(g, _):
                off = g * 16
                for c in range(CG):
                    acc_v[c, pl.ds(off, 16)] = neginf
                flg_v[g] = 0
                return 0

            lax.fori_loop(0, GROUPS, init, 0)

            def cams(n, _):
                cam(n, b, cg, p0)
                return 0

            lax.fori_loop(0, N, cams, 0)

            def finish(g, _):
                off = g * 16
                zero_or_ninf = jnp.where(flg_v[g] > 0,
                                         jnp.float32(0.0),
                                         jnp.float32(-jnp.inf))
                for c in range(CG):
                    acc_v[c, pl.ds(off, 16)] = jnp.maximum(
                        acc_v[c, pl.ds(off, 16)], zero_or_ninf)
                return 0

            lax.fori_loop(0, GROUPS, finish, 0)
            pltpu.sync_copy(
                acc_v, out_hbm.at[b, pl.ds(cg * CG, CG), pl.ds(p0, PPTS)])
            return 0

        lax.fori_loop(0, TPW, task_body, 0)

    return k(images_flat, px, py)


def _rotation_from_euler(rolls, pitchs, yaws):
    si, sj, sk = jnp.sin(rolls), jnp.sin(pitchs), jnp.sin(yaws)
    ci, cj, ck = jnp.cos(rolls), jnp.cos(pitchs), jnp.cos(yaws)
    cc, cs = ci * ck, ci * sk
    sc, ss = si * ck, si * sk
    zeros = jnp.zeros_like(si)
    ones = jnp.ones_like(si)
    row0 = jnp.stack([cj * ck, sj * sc - cs, sj * cc + ss, zeros], axis=-1)
    row1 = jnp.stack([cj * sk, sj * ss + cc, sj * cs - sc, zeros], axis=-1)
    row2 = jnp.stack([-sj, cj * si, cj * ci, zeros], axis=-1)
    row3 = jnp.stack([zeros, zeros, zeros, ones], axis=-1)
    return jnp.stack([row0, row1, row2, row3], axis=1)


def _pix_coords(Ks, RTs, translation, yaw_roll_pitch):
    zs = translation[:, 2]
    rolls = yaw_roll_pitch[:, 1]
    pitchs = yaw_roll_pitch[:, 2]
    xl = jnp.linspace(-60.0, 60.0, 200)
    yl = jnp.linspace(-30.0, 30.0, 100)
    Yg, Xg = jnp.meshgrid(yl, xl, indexing='ij')
    x = jnp.broadcast_to(Xg.flatten()[None, :], (B, PTS))
    y = jnp.broadcast_to(Yg.flatten()[None, :], (B, PTS))
    z = jnp.ones_like(x) * zs[:, None]
    d = jnp.ones_like(x)
    coords = jnp.stack([x, y, z, d], axis=1)
    R = _rotation_from_euler(pitchs, rolls, jnp.zeros_like(rolls))
    planes = R @ coords
    planes = jnp.tile(planes, (N, 1, 1))
    P = (Ks @ RTs).reshape(-1, 4, 4)
    pix = P @ planes
    py = pix[:, 0] + H / 2.0
    px = pix[:, 2] - W / 8.0
    return px, py


def kernel(images, Ks, RTs, translation, yaw_roll_pitch):
    px, py = _pix_coords(Ks, RTs, translation, yaw_roll_pitch)
    px = jnp.pad(px, ((0, 0), (0, PTSP - PTS)),
                 constant_values=1.0e9).reshape(-1)
    py = jnp.pad(py, ((0, 0), (0, PTSP - PTS)),
                 constant_values=1.0e9).reshape(-1)
    images_flat = images.reshape(-1)
    out = _sc_warp(images_flat, px, py)
    return out[:, :, :PTS].reshape(B, C, 100, 200)


# speedup vs baseline: 1.1205x; 1.0687x over previous
import functools

import jax
import jax.numpy as jnp
from jax import lax
from jax.experimental import pallas as pl
from jax.experimental.pallas import tpu as pltpu
from jax.experimental.pallas import tpu_sc as plsc

B, N, C, H, W = 4, 6, 64, 40, 80
HW = H * W
PTS = 20000
PTSP = 20000
NWARP = B * N
NCORE, NSUB = 2, 16
NWORK = NCORE * NSUB
CG = 8
NCG = C // CG
PARTS = 2
PPTS = PTSP // PARTS
GROUPS = PPTS // 16
NTASK = B * NCG * PARTS
TPW = NTASK // NWORK


def _sc_warp(images_flat, px, py):
    mesh = plsc.VectorSubcoreMesh(core_axis_name="c", subcore_axis_name="s")

    @functools.partial(
        pl.kernel,
        out_type=jax.ShapeDtypeStruct((B, C, PTSP), jnp.float32),
        mesh=mesh,
        compiler_params=pltpu.CompilerParams(
            use_tc_tiling_on_sc=False, needs_layout_passes=False,
            disable_bounds_checks=True),
        scratch_types=[
            pltpu.VMEM((CG * HW,), jnp.float32),
            pltpu.VMEM((CG, PPTS), jnp.float32),
            pltpu.VMEM((PPTS,), jnp.float32),
            pltpu.VMEM((PPTS,), jnp.float32),
            pltpu.SMEM((GROUPS,), jnp.int32),
        ],
    )
    def k(img_hbm, px_hbm, py_hbm, out_hbm, img_v, acc_v, px_v, py_v, flg_v):
        wid = lax.axis_index("s") * NCORE + lax.axis_index("c")
        neginf = jnp.full((16,), -jnp.inf, jnp.float32)

        def cam(n, b, cg, p0):
            wrp = b * N + n
            pltpu.sync_copy(
                img_hbm.at[pl.ds((wrp * C + cg * CG) * HW, CG * HW)], img_v)
            pltpu.sync_copy(px_hbm.at[pl.ds(wrp * PTSP + p0, PPTS)], px_v)
            pltpu.sync_copy(py_hbm.at[pl.ds(wrp * PTSP + p0, PPTS)], py_v)

            def group(g, _):
                off = g * 16
                pxv = jnp.clip(px_v[pl.ds(off, 16)], -4.0e4, 4.0e4)
                pyv = jnp.clip(py_v[pl.ds(off, 16)], -4.0e4, 4.0e4)
                xt = pxv.astype(jnp.int32)
                yt = pyv.astype(jnp.int32)
                x0 = jnp.where(pxv < xt.astype(jnp.float32), xt - 1, xt)
                y0 = jnp.where(pyv < yt.astype(jnp.float32), yt - 1, yt)
                inb = (x0 >= 0) & (x0 <= W - 2) & (y0 >= 0) & (y0 <= H - 2)

                def sample(_):
                    x0f = x0.astype(jnp.float32)
                    y0f = y0.astype(jnp.float32)
                    wx1 = pxv - x0f
                    wx0 = (x0f + 1.0) - pxv
                    m = jnp.where(inb, 1.0, 0.0).astype(jnp.float32)
                    wy1 = (pyv - y0f) * m
                    wy0 = ((y0f + 1.0) - pyv) * m
                    w00 = wx0 * wy0
                    w01 = wx0 * wy1
                    w10 = wx1 * wy0
                    w11 = wx1 * wy1
                    base = jnp.where(inb, y0 * W + x0, 0)
                    b01 = base + W
                    b10 = base + 1
                    b11 = base + W + 1
                    for c in range(CG):
                        coff = c * HW
                        v00 = plsc.load_gather(img_v, [base + coff])
                        v01 = plsc.load_gather(img_v, [b01 + coff])
                        v10 = plsc.load_gather(img_v, [b10 + coff])
                        v11 = plsc.load_gather(img_v, [b11 + coff])
                        val = (w00 * v00 + w01 * v01) + (w10 * v10 + w11 * v11)
                        acc_v[c, pl.ds(off, 16)] = jnp.maximum(
                            acc_v[c, pl.ds(off, 16)], val)
                    return 0

                def skip(_):
                    flg_v[g] = 1
                    return 0

                lax.cond(jnp.any(inb), sample, skip, 0)
                return 0

            lax.fori_loop(0, GROUPS, group, 0)

        def task_body(t, _):
            tid = wid * TPW + t
            part = lax.rem(tid, PARTS)
            rest = lax.div(tid, PARTS)
            cg = lax.rem(rest, NCG)
            b = lax.div(rest, NCG)
            p0 = part * PPTS

            def init(g, _):
                off = g * 16
                for c in range(CG):
                    acc_v[c, pl.ds(off, 16)] = neginf
                flg_v[g] = 0
                return 0

            lax.fori_loop(0, GROUPS, init, 0)

            def cams(n, _):
                cam(n, b, cg, p0)
                return 0

            lax.fori_loop(0, N, cams, 0)

            def finish(g, _):
                off = g * 16
                zero_or_ninf = jnp.where(flg_v[g] > 0,
                                         jnp.float32(0.0),
                                         jnp.float32(-jnp.inf))
                for c in range(CG):
                    acc_v[c, pl.ds(off, 16)] = jnp.maximum(
                        acc_v[c, pl.ds(off, 16)], zero_or_ninf)
                return 0

            lax.fori_loop(0, GROUPS, finish, 0)
            pltpu.sync_copy(
                acc_v, out_hbm.at[b, pl.ds(cg * CG, CG), pl.ds(p0, PPTS)])
            return 0

        lax.fori_loop(0, TPW, task_body, 0)

    return k(images_flat, px, py)


def _rotation_from_euler(rolls, pitchs, yaws):
    si, sj, sk = jnp.sin(rolls), jnp.sin(pitchs), jnp.sin(yaws)
    ci, cj, ck = jnp.cos(rolls), jnp.cos(pitchs), jnp.cos(yaws)
    cc, cs = ci * ck, ci * sk
    sc, ss = si * ck, si * sk
    zeros = jnp.zeros_like(si)
    ones = jnp.ones_like(si)
    row0 = jnp.stack([cj * ck, sj * sc - cs, sj * cc + ss, zeros], axis=-1)
    row1 = jnp.stack([cj * sk, sj * ss + cc, sj * cs - sc, zeros], axis=-1)
    row2 = jnp.stack([-sj, cj * si, cj * ci, zeros], axis=-1)
    row3 = jnp.stack([zeros, zeros, zeros, ones], axis=-1)
    return jnp.stack([row0, row1, row2, row3], axis=1)


def _pix_coords(Ks, RTs, translation, yaw_roll_pitch):
    zs = translation[:, 2]
    rolls = yaw_roll_pitch[:, 1]
    pitchs = yaw_roll_pitch[:, 2]
    xl = jnp.linspace(-60.0, 60.0, 200)
    yl = jnp.linspace(-30.0, 30.0, 100)
    Yg, Xg = jnp.meshgrid(yl, xl, indexing='ij')
    x = jnp.broadcast_to(Xg.flatten()[None, :], (B, PTS))
    y = jnp.broadcast_to(Yg.flatten()[None, :], (B, PTS))
    z = jnp.ones_like(x) * zs[:, None]
    d = jnp.ones_like(x)
    coords = jnp.stack([x, y, z, d], axis=1)
    R = _rotation_from_euler(pitchs, rolls, jnp.zeros_like(rolls))
    planes = R @ coords
    planes = jnp.tile(planes, (N, 1, 1))
    P = (Ks @ RTs).reshape(-1, 4, 4)
    pix = P @ planes
    py = pix[:, 0] + H / 2.0
    px = pix[:, 2] - W / 8.0
    return px, py


def kernel(images, Ks, RTs, translation, yaw_roll_pitch):
    px, py = _pix_coords(Ks, RTs, translation, yaw_roll_pitch)
    images_flat = images.reshape(-1)
    out = _sc_warp(images_flat, px.reshape(-1), py.reshape(-1))
    return out.reshape(B, C, 100, 200)


# speedup vs baseline: 1.2063x; 1.0765x over previous
import functools

import jax
import jax.numpy as jnp
from jax import lax
from jax.experimental import pallas as pl
from jax.experimental.pallas import tpu as pltpu
from jax.experimental.pallas import tpu_sc as plsc

B, N, C, H, W = 4, 6, 64, 40, 80
HW = H * W
PTS = 20000
PTSP = 20000
NWARP = B * N
NCORE, NSUB = 2, 16
NWORK = NCORE * NSUB
CG = 8
NCG = C // CG
PARTS = 5
PPTS = PTSP // PARTS
GROUPS = PPTS // 16
NTASK = B * NCG * PARTS
TPW = NTASK // NWORK


def _sc_warp(images_flat, px, py):
    mesh = plsc.VectorSubcoreMesh(core_axis_name="c", subcore_axis_name="s")

    @functools.partial(
        pl.kernel,
        out_type=jax.ShapeDtypeStruct((B, C, PTSP), jnp.float32),
        mesh=mesh,
        compiler_params=pltpu.CompilerParams(
            use_tc_tiling_on_sc=False, needs_layout_passes=False,
            disable_bounds_checks=True),
        scratch_types=[
            pltpu.VMEM((CG * HW,), jnp.float32),
            pltpu.VMEM((CG * HW,), jnp.float32),
            pltpu.VMEM((PPTS,), jnp.float32),
            pltpu.VMEM((PPTS,), jnp.float32),
            pltpu.VMEM((PPTS,), jnp.float32),
            pltpu.VMEM((PPTS,), jnp.float32),
            pltpu.VMEM((CG, PPTS), jnp.float32),
            pltpu.SMEM((GROUPS,), jnp.int32),
            pltpu.SemaphoreType.DMA,
            pltpu.SemaphoreType.DMA,
        ],
    )
    def k(img_hbm, px_hbm, py_hbm, out_hbm,
          img0, img1, px0, px1, py0, py1, acc_v, flg_v, sem0, sem1):
        wid = lax.axis_index("s") * NCORE + lax.axis_index("c")
        neginf = jnp.full((16,), -jnp.inf, jnp.float32)
        bufs = [(img0, px0, py0, sem0), (img1, px1, py1, sem1)]

        def issue(n, b, cg, p0, slot):
            img_v, px_v, py_v, sem = bufs[slot]
            wrp = b * N + n
            return (
                pltpu.async_copy(
                    img_hbm.at[pl.ds((wrp * C + cg * CG) * HW, CG * HW)],
                    img_v, sem),
                pltpu.async_copy(
                    px_hbm.at[pl.ds(wrp * PTSP + p0, PPTS)], px_v, sem),
                pltpu.async_copy(
                    py_hbm.at[pl.ds(wrp * PTSP + p0, PPTS)], py_v, sem),
            )

        def cam(slot):
            img_v, px_v, py_v, _ = bufs[slot]

            def group(g, _):
                off = g * 16
                pxv = jnp.clip(px_v[pl.ds(off, 16)], -4.0e4, 4.0e4)
                pyv = jnp.clip(py_v[pl.ds(off, 16)], -4.0e4, 4.0e4)
                xt = pxv.astype(jnp.int32)
                yt = pyv.astype(jnp.int32)
                x0 = jnp.where(pxv < xt.astype(jnp.float32), xt - 1, xt)
                y0 = jnp.where(pyv < yt.astype(jnp.float32), yt - 1, yt)
                inb = (x0 >= 0) & (x0 <= W - 2) & (y0 >= 0) & (y0 <= H - 2)

                def sample(_):
                    x0f = x0.astype(jnp.float32)
                    y0f = y0.astype(jnp.float32)
                    wx1 = pxv - x0f
                    wx0 = (x0f + 1.0) - pxv
                    m = jnp.where(inb, 1.0, 0.0).astype(jnp.float32)
                    wy1 = (pyv - y0f) * m
                    wy0 = ((y0f + 1.0) - pyv) * m
                    w00 = wx0 * wy0
                    w01 = wx0 * wy1
                    w10 = wx1 * wy0
                    w11 = wx1 * wy1
                    base = jnp.where(inb, y0 * W + x0, 0)
                    b01 = base + W
                    b10 = base + 1
                    b11 = base + W + 1
                    for c in range(CG):
                        coff = c * HW
                        v00 = plsc.load_gather(img_v, [base + coff])
                        v01 = plsc.load_gather(img_v, [b01 + coff])
                        v10 = plsc.load_gather(img_v, [b10 + coff])
                        v11 = plsc.load_gather(img_v, [b11 + coff])
                        val = (w00 * v00 + w01 * v01) + (w10 * v10 + w11 * v11)
                        acc_v[c, pl.ds(off, 16)] = jnp.maximum(
                            acc_v[c, pl.ds(off, 16)], val)
                    return 0

                def skip(_):
                    flg_v[g] = 1
                    return 0

                lax.cond(jnp.any(inb), sample, skip, 0)
                return 0

            lax.fori_loop(0, GROUPS, group, 0)

        def task_body(t, _):
            tid = wid * TPW + t
            part = lax.rem(tid, PARTS)
            rest = lax.div(tid, PARTS)
            cg = lax.rem(rest, NCG)
            b = lax.div(rest, NCG)
            p0 = part * PPTS

            def init(g, _):
                off = g * 16
                for c in range(CG):
                    acc_v[c, pl.ds(off, 16)] = neginf
                flg_v[g] = 0
                return 0

            descs = issue(0, b, cg, p0, 0)
            lax.fori_loop(0, GROUPS, init, 0)

            for n in range(N):
                for d in descs:
                    d.wait()
                slot = n % 2
                if n + 1 < N:
                    descs = issue(n + 1, b, cg, p0, (n + 1) % 2)
                cam(slot)

            def finish(g, _):
                off = g * 16
                zero_or_ninf = jnp.where(flg_v[g] > 0,
                                         jnp.float32(0.0),
                                         jnp.float32(-jnp.inf))
                for c in range(CG):
                    acc_v[c, pl.ds(off, 16)] = jnp.maximum(
                        acc_v[c, pl.ds(off, 16)], zero_or_ninf)
                return 0

            lax.fori_loop(0, GROUPS, finish, 0)
            pltpu.sync_copy(
                acc_v, out_hbm.at[b, pl.ds(cg * CG, CG), pl.ds(p0, PPTS)])
            return 0

        lax.fori_loop(0, TPW, task_body, 0)

    return k(images_flat, px, py)


def _rotation_from_euler(rolls, pitchs, yaws):
    si, sj, sk = jnp.sin(rolls), jnp.sin(pitchs), jnp.sin(yaws)
    ci, cj, ck = jnp.cos(rolls), jnp.cos(pitchs), jnp.cos(yaws)
    cc, cs = ci * ck, ci * sk
    sc, ss = si * ck, si * sk
    zeros = jnp.zeros_like(si)
    ones = jnp.ones_like(si)
    row0 = jnp.stack([cj * ck, sj * sc - cs, sj * cc + ss, zeros], axis=-1)
    row1 = jnp.stack([cj * sk, sj * ss + cc, sj * cs - sc, zeros], axis=-1)
    row2 = jnp.stack([-sj, cj * si, cj * ci, zeros], axis=-1)
    row3 = jnp.stack([zeros, zeros, zeros, ones], axis=-1)
    return jnp.stack([row0, row1, row2, row3], axis=1)


def _pix_coords(Ks, RTs, translation, yaw_roll_pitch):
    zs = translation[:, 2]
    rolls = yaw_roll_pitch[:, 1]
    pitchs = yaw_roll_pitch[:, 2]
    xl = jnp.linspace(-60.0, 60.0, 200)
    yl = jnp.linspace(-30.0, 30.0, 100)
    Yg, Xg = jnp.meshgrid(yl, xl, indexing='ij')
    x = jnp.broadcast_to(Xg.flatten()[None, :], (B, PTS))
    y = jnp.broadcast_to(Yg.flatten()[None, :], (B, PTS))
    z = jnp.ones_like(x) * zs[:, None]
    d = jnp.ones_like(x)
    coords = jnp.stack([x, y, z, d], axis=1)
    R = _rotation_from_euler(pitchs, rolls, jnp.zeros_like(rolls))
    planes = R @ coords
    planes = jnp.tile(planes, (N, 1, 1))
    P = (Ks @ RTs).reshape(-1, 4, 4)
    pix = P @ planes
    py = pix[:, 0] + H / 2.0
    px = pix[:, 2] - W / 8.0
    return px, py


def kernel(images, Ks, RTs, translation, yaw_roll_pitch):
    px, py = _pix_coords(Ks, RTs, translation, yaw_roll_pitch)
    images_flat = images.reshape(-1)
    out = _sc_warp(images_flat, px.reshape(-1), py.reshape(-1))
    return out.reshape(B, C, 100, 200)
